# Initial kernel scaffold; baseline (speedup 1.0000x reference)
import jax
import jax.numpy as jnp
from jax.experimental import pallas as pl


def kernel(x, edge_index):
    raise NotImplementedError("write your pallas kernel here")


# speedup vs baseline: 3.1661x; 3.1661x over previous
import functools

import jax
import jax.numpy as jnp
from jax import lax
from jax.experimental import pallas as pl
from jax.experimental.pallas import tpu as pltpu
from jax.experimental.pallas import tpu_sc as plsc

NC = 2
NS = 16
NW = NC * NS

N_NODES = 10000
D = 128
CHUNK = 128
GROUP = 8
N_GROUPS = 10
CHUNKS_PER_TILE = GROUP * N_GROUPS
E_PER_TILE = CHUNKS_PER_TILE * CHUNK
E_PAD = NW * E_PER_TILE
ACC_ROWS = 10112
ROWS_PER_TILE = ACC_ROWS // NS


def _sc_body(x_hbm, src_hbm, dst_hbm, zero_hbm, out_hbm,
             acc, sidx, didx, rows, sem):
    c = lax.axis_index("c")
    s = lax.axis_index("s")
    wid = s * NC + c

    r0 = s * ROWS_PER_TILE
    pltpu.sync_copy(zero_hbm.at[pl.ds(r0, ROWS_PER_TILE)],
                    acc.at[pl.ds(r0, ROWS_PER_TILE)])
    plsc.subcore_barrier()

    chunk_base = wid * CHUNKS_PER_TILE

    @pl.loop(0, N_GROUPS)
    def _group(g):
        row = chunk_base + g * GROUP
        pltpu.sync_copy(src_hbm.at[pl.ds(row, GROUP)], sidx)
        pltpu.sync_copy(dst_hbm.at[pl.ds(row, GROUP)], didx)
        for j in range(GROUP):
            pltpu.async_copy(x_hbm.at[sidx.at[j]], rows, sem).wait()
            pltpu.sync_copy(rows, acc.at[didx.at[j]], add=True)

    plsc.subcore_barrier()

    pltpu.sync_copy(acc.at[pl.ds(r0, ROWS_PER_TILE)],
                    out_hbm.at[c, pl.ds(r0, ROWS_PER_TILE)])


def _combine_body(x_ref, p0_ref, p1_ref, o_ref):
    o_ref[...] = x_ref[...] + p0_ref[...] + p1_ref[...]


@jax.jit
def kernel(x, edge_index):
    src = edge_index[0]
    dst = edge_index[1]
    n_extra = E_PAD - src.shape[0]
    pad_src = jnp.zeros((n_extra,), jnp.int32)
    pad_dst = N_NODES + (jnp.arange(n_extra, dtype=jnp.int32) % NS)
    src_p = jnp.concatenate([src, pad_src]).reshape(E_PAD // CHUNK, CHUNK)
    dst_p = jnp.concatenate([dst, pad_dst]).reshape(E_PAD // CHUNK, CHUNK)
    zeros = jnp.zeros((ACC_ROWS, D), jnp.float32)

    mesh = plsc.VectorSubcoreMesh(
        core_axis_name="c", subcore_axis_name="s",
        num_cores=NC, num_subcores=NS)

    partials = pl.kernel(
        _sc_body,
        out_type=jax.ShapeDtypeStruct((NC, ACC_ROWS, D), jnp.float32),
        mesh=mesh,
        scratch_types=[
            pltpu.VMEM_SHARED((ACC_ROWS, D), jnp.float32),
            pltpu.VMEM((GROUP, CHUNK), jnp.int32),
            pltpu.VMEM((GROUP, CHUNK), jnp.int32),
            pltpu.VMEM((CHUNK, D), jnp.float32),
            pltpu.SemaphoreType.DMA,
        ],
    )(x, src_p, dst_p, zeros)

    blk = 1000
    out = pl.pallas_call(
        _combine_body,
        grid=(N_NODES // blk,),
        in_specs=[pl.BlockSpec((blk, D), lambda i: (i, 0))] * 3,
        out_specs=pl.BlockSpec((blk, D), lambda i: (i, 0)),
        out_shape=jax.ShapeDtypeStruct((N_NODES, D), jnp.float32),
    )(x, partials[0, :N_NODES], partials[1, :N_NODES])
    return out


# speedup vs baseline: 3.4727x; 1.0968x over previous
import functools

import jax
import jax.numpy as jnp
from jax import lax
from jax.experimental import pallas as pl
from jax.experimental.pallas import tpu as pltpu
from jax.experimental.pallas import tpu_sc as plsc

NC = 2
NS = 16
NW = NC * NS

N_NODES = 10000
D = 128
CHUNK = 128
GROUP = 16
N_GROUPS = 5
CHUNKS_PER_TILE = GROUP * N_GROUPS
E_PER_TILE = CHUNKS_PER_TILE * CHUNK
E_PAD = NW * E_PER_TILE
ACC_ROWS = 10112
ROWS_PER_TILE = ACC_ROWS // NS


def _sc_body(x_hbm, src_hbm, dst_hbm, zero_hbm, out_hbm,
             acc, sidx, didx, rows0, rows1, sem0, sem1):
    c = lax.axis_index("c")
    s = lax.axis_index("s")
    wid = s * NC + c

    r0 = s * ROWS_PER_TILE
    pltpu.sync_copy(zero_hbm.at[pl.ds(r0, ROWS_PER_TILE)],
                    acc.at[pl.ds(r0, ROWS_PER_TILE)])
    plsc.subcore_barrier()

    chunk_base = wid * CHUNKS_PER_TILE
    rows = (rows0, rows1)
    sems = (sem0, sem1)

    @pl.loop(0, N_GROUPS)
    def _group(g):
        row = chunk_base + g * GROUP
        pltpu.sync_copy(src_hbm.at[pl.ds(row, GROUP)], sidx)
        pltpu.sync_copy(dst_hbm.at[pl.ds(row, GROUP)], didx)
        cps = [None] * GROUP
        cps[0] = pltpu.async_copy(x_hbm.at[sidx.at[0]], rows[0], sems[0])
        for j in range(GROUP):
            if j + 1 < GROUP:
                b = (j + 1) % 2
                cps[j + 1] = pltpu.async_copy(
                    x_hbm.at[sidx.at[j + 1]], rows[b], sems[b])
            cps[j].wait()
            pltpu.sync_copy(rows[j % 2], acc.at[didx.at[j]], add=True)

    plsc.subcore_barrier()

    pltpu.sync_copy(acc.at[pl.ds(r0, ROWS_PER_TILE)],
                    out_hbm.at[c, pl.ds(r0, ROWS_PER_TILE)])


def _combine_body(x_ref, p0_ref, p1_ref, o_ref):
    o_ref[...] = x_ref[...] + p0_ref[...] + p1_ref[...]


@jax.jit
def kernel(x, edge_index):
    src = edge_index[0]
    dst = edge_index[1]
    n_extra = E_PAD - src.shape[0]
    pad_src = jnp.zeros((n_extra,), jnp.int32)
    pad_dst = N_NODES + (jnp.arange(n_extra, dtype=jnp.int32) % NS)
    src_p = jnp.concatenate([src, pad_src]).reshape(E_PAD // CHUNK, CHUNK)
    dst_p = jnp.concatenate([dst, pad_dst]).reshape(E_PAD // CHUNK, CHUNK)
    zeros = jnp.zeros((ACC_ROWS, D), jnp.float32)

    mesh = plsc.VectorSubcoreMesh(
        core_axis_name="c", subcore_axis_name="s",
        num_cores=NC, num_subcores=NS)

    partials = pl.kernel(
        _sc_body,
        out_type=jax.ShapeDtypeStruct((NC, ACC_ROWS, D), jnp.float32),
        mesh=mesh,
        scratch_types=[
            pltpu.VMEM_SHARED((ACC_ROWS, D), jnp.float32),
            pltpu.VMEM((GROUP, CHUNK), jnp.int32),
            pltpu.VMEM((GROUP, CHUNK), jnp.int32),
            pltpu.VMEM((CHUNK, D), jnp.float32),
            pltpu.VMEM((CHUNK, D), jnp.float32),
            pltpu.SemaphoreType.DMA,
            pltpu.SemaphoreType.DMA,
        ],
    )(x, src_p, dst_p, zeros)

    blk = 1000
    out = pl.pallas_call(
        _combine_body,
        grid=(N_NODES // blk,),
        in_specs=[pl.BlockSpec((blk, D), lambda i: (i, 0))] * 3,
        out_specs=pl.BlockSpec((blk, D), lambda i: (i, 0)),
        out_shape=jax.ShapeDtypeStruct((N_NODES, D), jnp.float32),
    )(x, partials[0, :N_NODES], partials[1, :N_NODES])
    return out


# speedup vs baseline: 10.3269x; 2.9737x over previous
import jax
import jax.numpy as jnp
from jax import lax
from jax.experimental import pallas as pl
from jax.experimental.pallas import tpu as pltpu
from jax.experimental.pallas import tpu_sc as plsc

NC = 2
NS = 16
NW = NC * NS
N_NODES = 10000
D = 128

CHUNK = 128
GROUP = 4
N_CHUNKS = 80
N_STAGE = N_CHUNKS // GROUP
E_PER_TILE = N_CHUNKS * CHUNK
E_PAD = NW * E_PER_TILE
RING = 3
AHEAD = 2

XP_ROWS = N_NODES + 8


def _sc_body(xp_hbm, src_hbm, dst_hbm, zero_hbm, out_hbm,
             acc, sidx, didx, rows0, rows1, rows2, sem0, sem1, sem2):
    c = lax.axis_index("c")
    s = lax.axis_index("s")
    wid = s * NC + c

    @pl.when(s < 15)
    def _():
        pltpu.sync_copy(zero_hbm.at[pl.ds(s * 632, 632)],
                        acc.at[pl.ds(s * 632, 632)])

    @pl.when(s == 15)
    def _():
        pltpu.sync_copy(zero_hbm.at[pl.ds(9480, 520)],
                        acc.at[pl.ds(9480, 520)])

    plsc.subcore_barrier()

    rows = (rows0, rows1, rows2)
    sems = (sem0, sem1, sem2)
    chunk_base = wid * N_CHUNKS

    cps = [None] * N_CHUNKS
    for i in range(-AHEAD, N_CHUNKS):
        f = i + AHEAD
        if f < N_CHUNKS:
            if f % GROUP == 0:
                g = f // GROUP
                row = chunk_base + f
                pltpu.sync_copy(src_hbm.at[pl.ds(row, GROUP)], sidx)
                pltpu.sync_copy(dst_hbm.at[pl.ds(row, GROUP)],
                                didx.at[g % 2])
            b = f % RING
            cps[f] = pltpu.async_copy(
                xp_hbm.at[sidx.at[f % GROUP]], rows[b], sems[b])
        if i >= 0:
            cps[i].wait()
            pltpu.sync_copy(rows[i % RING],
                            acc.at[didx.at[(i // GROUP) % 2, i % GROUP]],
                            add=True)

    plsc.subcore_barrier()

    @pl.when(s < 15)
    def _():
        pltpu.sync_copy(acc.at[pl.ds(s * 632, 632)],
                        out_hbm.at[c, pl.ds(s * 632, 632)])

    @pl.when(s == 15)
    def _():
        pltpu.sync_copy(acc.at[pl.ds(9480, 520)],
                        out_hbm.at[c, pl.ds(9480, 520)])


def _combine_body(x_ref, p0_ref, p1_ref, o_ref):
    o_ref[...] = x_ref[...] + p0_ref[...] + p1_ref[...]


@jax.jit
def kernel(x, edge_index):
    src = edge_index[0]
    dst = edge_index[1]
    n_extra = E_PAD - src.shape[0]
    pad_src = N_NODES + (jnp.arange(n_extra, dtype=jnp.int32) % 8)
    pad_dst = jnp.zeros((n_extra,), jnp.int32)
    src_p = jnp.concatenate([src, pad_src]).reshape(E_PAD // CHUNK, CHUNK)
    dst_p = jnp.concatenate([dst, pad_dst]).reshape(E_PAD // CHUNK, CHUNK)
    xp = jnp.concatenate([x, jnp.zeros((XP_ROWS - N_NODES, D), jnp.float32)])
    zeros = jnp.zeros((N_NODES, D), jnp.float32)

    mesh = plsc.VectorSubcoreMesh(
        core_axis_name="c", subcore_axis_name="s",
        num_cores=NC, num_subcores=NS)

    partials = pl.kernel(
        _sc_body,
        out_type=jax.ShapeDtypeStruct((NC, N_NODES, D), jnp.float32),
        mesh=mesh,
        scratch_types=[
            pltpu.VMEM_SHARED((N_NODES, D), jnp.float32),
            pltpu.VMEM((GROUP, CHUNK), jnp.int32),
            pltpu.VMEM((2, GROUP, CHUNK), jnp.int32),
            pltpu.VMEM((CHUNK, D), jnp.float32),
            pltpu.VMEM((CHUNK, D), jnp.float32),
            pltpu.VMEM((CHUNK, D), jnp.float32),
            pltpu.SemaphoreType.DMA,
            pltpu.SemaphoreType.DMA,
            pltpu.SemaphoreType.DMA,
        ],
    )(xp, src_p, dst_p, zeros)

    blk = 1000
    out = pl.pallas_call(
        _combine_body,
        grid=(N_NODES // blk,),
        in_specs=[pl.BlockSpec((blk, D), lambda i: (i, 0))] * 3,
        out_specs=pl.BlockSpec((blk, D), lambda i: (i, 0)),
        out_shape=jax.ShapeDtypeStruct((N_NODES, D), jnp.float32),
    )(x, partials[0], partials[1])
    return out


# speedup vs baseline: 14.3161x; 1.3863x over previous
import jax
import jax.numpy as jnp
from jax import lax
from jax.experimental import pallas as pl
from jax.experimental.pallas import tpu as pltpu
from jax.experimental.pallas import tpu_sc as plsc

NC = 2
NS = 16
NW = NC * NS
N_NODES = 10000
D = 128

CHUNK = 128
N_CHUNK_ROWS = 2500
BASE_CHUNKS = 76
GROUP = 4
RING = 3
AHEAD = 2
N_EXTRA = N_CHUNK_ROWS - NW * BASE_CHUNKS
EXTRA_TILES = N_EXTRA // GROUP


def _sc_body(x_hbm, e_hbm, out_hbm, acc, estage, didx, rows0, rows1, rows2,
             gs0, gs1, gs2, ss0, ss1, ss2):
    c = lax.axis_index("c")
    s = lax.axis_index("s")
    wid = s * NC + c

    @pl.when(s < 15)
    def _():
        pltpu.sync_copy(x_hbm.at[pl.ds(s * 632, 632)],
                        acc.at[pl.ds(s * 632, 632)])

    @pl.when(s == 15)
    def _():
        pltpu.sync_copy(x_hbm.at[pl.ds(9480, 520)],
                        acc.at[pl.ds(9480, 520)])

    plsc.subcore_barrier()

    rows = (rows0, rows1, rows2)
    gsems = (gs0, gs1, gs2)
    ssems = (ss0, ss1, ss2)

    def run_chunks(chunk_base, n_chunks, gcps, scps):
        def fire_gather(f):
            b = f % RING
            if f >= RING:
                scps[f - RING].wait()
            if f % GROUP == 0:
                pltpu.sync_copy(e_hbm.at[:, pl.ds(chunk_base + f, GROUP)],
                                estage)
            for v in range(CHUNK // 16):
                didx[b, pl.ds(v * 16, 16)] = estage[1, f % GROUP,
                                                    pl.ds(v * 16, 16)]
            gcps[f] = pltpu.async_copy(
                x_hbm.at[estage.at[0, f % GROUP]], rows[b], gsems[b])

        def fire_scatter(i):
            b = i % RING
            gcps[i].wait()
            scps[i] = pltpu.async_copy(
                rows[b], acc.at[didx.at[b]], ssems[b], add=True)

        for i in range(-AHEAD, n_chunks):
            f = i + AHEAD
            if f < n_chunks:
                fire_gather(f)
            if i >= 0:
                fire_scatter(i)
        for i in range(max(0, n_chunks - RING), n_chunks):
            scps[i].wait()

    gcps = [None] * BASE_CHUNKS
    scps = [None] * BASE_CHUNKS
    run_chunks(wid * BASE_CHUNKS, BASE_CHUNKS, gcps, scps)

    @pl.when(wid < EXTRA_TILES)
    def _():
        gcps2 = [None] * GROUP
        scps2 = [None] * GROUP
        run_chunks(NW * BASE_CHUNKS + wid * GROUP, GROUP, gcps2, scps2)

    plsc.subcore_barrier()

    @pl.when(s < 15)
    def _():
        pltpu.sync_copy(acc.at[pl.ds(s * 632, 632)],
                        out_hbm.at[c, pl.ds(s * 632, 632)])

    @pl.when(s == 15)
    def _():
        pltpu.sync_copy(acc.at[pl.ds(9480, 520)],
                        out_hbm.at[c, pl.ds(9480, 520)])


def _combine_body(x_ref, p0_ref, p1_ref, o_ref):
    o_ref[...] = p0_ref[0] + p1_ref[0] - x_ref[...]


@jax.jit
def kernel(x, edge_index):
    e3 = edge_index.reshape(NC, N_CHUNK_ROWS, CHUNK)

    mesh = plsc.VectorSubcoreMesh(
        core_axis_name="c", subcore_axis_name="s",
        num_cores=NC, num_subcores=NS)

    partials = pl.kernel(
        _sc_body,
        out_type=jax.ShapeDtypeStruct((NC, N_NODES, D), jnp.float32),
        mesh=mesh,
        scratch_types=[
            pltpu.VMEM_SHARED((N_NODES, D), jnp.float32),
            pltpu.VMEM((2, GROUP, CHUNK), jnp.int32),
            pltpu.VMEM((RING, CHUNK), jnp.int32),
            pltpu.VMEM((CHUNK, D), jnp.float32),
            pltpu.VMEM((CHUNK, D), jnp.float32),
            pltpu.VMEM((CHUNK, D), jnp.float32),
            pltpu.SemaphoreType.DMA,
            pltpu.SemaphoreType.DMA,
            pltpu.SemaphoreType.DMA,
            pltpu.SemaphoreType.DMA,
            pltpu.SemaphoreType.DMA,
            pltpu.SemaphoreType.DMA,
        ],
    )(x, e3)

    blk = 2000
    out = pl.pallas_call(
        _combine_body,
        grid=(N_NODES // blk,),
        in_specs=[
            pl.BlockSpec((blk, D), lambda i: (i, 0)),
            pl.BlockSpec((1, blk, D), lambda i: (0, i, 0)),
            pl.BlockSpec((1, blk, D), lambda i: (1, i, 0)),
        ],
        out_specs=pl.BlockSpec((blk, D), lambda i: (i, 0)),
        out_shape=jax.ShapeDtypeStruct((N_NODES, D), jnp.float32),
    )(x, partials, partials)
    return out
